# baseline (speedup 1.0000x reference)
import functools

import jax
import jax.numpy as jnp
from jax.experimental import pallas as pl
from jax.experimental.pallas import tpu as pltpu

_EPS = 1e-5


def _rot(x, k):
    return jnp.concatenate([x[:, k:], x[:, :k]], axis=1)


def _stats_kernel(x_ref, mom_ref):
    j = pl.program_id(1)

    @pl.when(j == 0)
    def _():
        mom_ref[...] = jnp.zeros_like(mom_ref)

    nb = x_ref.shape[0]
    acc = None
    for b in range(nb):
        xb = x_ref[b]
        r0 = jnp.sum(xb, axis=0, keepdims=True)
        r1 = jnp.sum(xb * xb, axis=0, keepdims=True)
        r2 = jnp.sum(xb * _rot(xb, 64), axis=0, keepdims=True)
        r3 = jnp.sum(xb * _rot(xb, 128), axis=0, keepdims=True)
        blk = jnp.concatenate([r0, r1, r2, r3], axis=0)
        acc = blk if acc is None else acc + blk
    mom_ref[0] += acc


def _factor_kernel(mom_ref, w_ref, b_ref, a_ref, beff_ref, *, n_samples):
    m = mom_ref[0] + mom_ref[1]
    inv_n = 1.0 / float(n_samples)

    def seg(row, q):
        return m[row:row + 1, q * 64:(q + 1) * 64] * inv_n

    mean = [seg(0, q) for q in range(4)]
    a = {}
    for q in range(4):
        a[(q, q)] = seg(1, q) - mean[q] * mean[q] + _EPS
    a[(0, 1)] = seg(2, 0) - mean[0] * mean[1]
    a[(1, 2)] = seg(2, 1) - mean[1] * mean[2]
    a[(2, 3)] = seg(2, 2) - mean[2] * mean[3]
    a[(0, 3)] = seg(2, 3) - mean[0] * mean[3]
    a[(0, 2)] = seg(3, 0) - mean[0] * mean[2]
    a[(1, 3)] = seg(3, 1) - mean[1] * mean[3]

    def A(i, j):
        return a[(min(i, j), max(i, j))]

    l = {}
    l[(0, 0)] = jnp.sqrt(A(0, 0))
    inv00 = 1.0 / l[(0, 0)]
    l[(1, 0)] = A(1, 0) * inv00
    l[(2, 0)] = A(2, 0) * inv00
    l[(3, 0)] = A(3, 0) * inv00
    l[(1, 1)] = jnp.sqrt(A(1, 1) - l[(1, 0)] ** 2)
    inv11 = 1.0 / l[(1, 1)]
    l[(2, 1)] = (A(2, 1) - l[(2, 0)] * l[(1, 0)]) * inv11
    l[(3, 1)] = (A(3, 1) - l[(3, 0)] * l[(1, 0)]) * inv11
    l[(2, 2)] = jnp.sqrt(A(2, 2) - l[(2, 0)] ** 2 - l[(2, 1)] ** 2)
    inv22 = 1.0 / l[(2, 2)]
    l[(3, 2)] = (A(3, 2) - l[(3, 0)] * l[(2, 0)] - l[(3, 1)] * l[(2, 1)]) * inv22
    l[(3, 3)] = jnp.sqrt(A(3, 3) - l[(3, 0)] ** 2 - l[(3, 1)] ** 2 - l[(3, 2)] ** 2)
    inv33 = 1.0 / l[(3, 3)]

    kk = {(0, 0): inv00, (1, 1): inv11, (2, 2): inv22, (3, 3): inv33}
    kk[(1, 0)] = -(l[(1, 0)] * kk[(0, 0)]) * inv11
    kk[(2, 0)] = -(l[(2, 0)] * kk[(0, 0)] + l[(2, 1)] * kk[(1, 0)]) * inv22
    kk[(2, 1)] = -(l[(2, 1)] * kk[(1, 1)]) * inv22
    kk[(3, 0)] = -(l[(3, 0)] * kk[(0, 0)] + l[(3, 1)] * kk[(1, 0)]
                   + l[(3, 2)] * kk[(2, 0)]) * inv33
    kk[(3, 1)] = -(l[(3, 1)] * kk[(1, 1)] + l[(3, 2)] * kk[(2, 1)]) * inv33
    kk[(3, 2)] = -(l[(3, 2)] * kk[(2, 2)]) * inv33

    mm = {}
    beffs = []
    for p in range(4):
        wp = w_ref[p]
        w = [wp[r:r + 1, :] for r in range(4)]
        bp = b_ref[p:p + 1, :]
        for s in range(4):
            acc = w[0] * kk[(s, 0)]
            for r in range(1, s + 1):
                acc = acc + w[r] * kk[(s, r)]
            mm[(p, s)] = acc
            bp = bp - acc * mean[s]
        beffs.append(bp)

    rows = [jnp.concatenate([mm[(p, (p + s) % 4)] for p in range(4)], axis=1)
            for s in range(4)]
    a_ref[...] = jnp.concatenate(rows, axis=0)
    beff_ref[...] = jnp.concatenate(beffs, axis=1)


def _apply_kernel(x_ref, a_ref, beff_ref, o_ref):
    nb = x_ref.shape[0]
    a0 = a_ref[0:1, :]
    a1 = a_ref[1:2, :]
    a2 = a_ref[2:3, :]
    a3 = a_ref[3:4, :]
    bb = beff_ref[0:1, :]
    for b in range(nb):
        xb = x_ref[b]
        acc = bb + a0 * xb
        acc = acc + a1 * _rot(xb, 64)
        acc = acc + a2 * _rot(xb, 128)
        acc = acc + a3 * _rot(xb, 192)
        o_ref[b] = acc


def kernel(x, weight, bias):
    B, C, H, W = x.shape
    HW = H * W
    N = B * HW
    xt = jnp.transpose(x, (0, 2, 3, 1)).reshape(B, HW, C)

    NSPLIT = 2
    BB_S = 4
    BB_A = 4
    steps_s = B // (BB_S * NSPLIT)
    steps_a = B // (BB_A * NSPLIT)

    moments = pl.pallas_call(
        _stats_kernel,
        grid=(NSPLIT, steps_s),
        in_specs=[pl.BlockSpec((BB_S, HW, C), lambda i, j: (i * steps_s + j, 0, 0))],
        out_specs=pl.BlockSpec((1, 4, C), lambda i, j: (i, 0, 0)),
        out_shape=jax.ShapeDtypeStruct((NSPLIT, 4, C), jnp.float32),
        compiler_params=pltpu.CompilerParams(
            dimension_semantics=("parallel", "arbitrary"),
            vmem_limit_bytes=56 * 1024 * 1024,
        ),
        name="qbn_stats",
    )(xt)

    amat, beff = pl.pallas_call(
        functools.partial(_factor_kernel, n_samples=N),
        grid=(1,),
        in_specs=[
            pl.BlockSpec((NSPLIT, 4, C), lambda i: (0, 0, 0)),
            pl.BlockSpec((4, 4, C // 4), lambda i: (0, 0, 0)),
            pl.BlockSpec((4, C // 4), lambda i: (0, 0)),
        ],
        out_specs=[
            pl.BlockSpec((4, C), lambda i: (0, 0)),
            pl.BlockSpec((1, C), lambda i: (0, 0)),
        ],
        out_shape=[
            jax.ShapeDtypeStruct((4, C), jnp.float32),
            jax.ShapeDtypeStruct((1, C), jnp.float32),
        ],
        name="qbn_factor",
    )(moments, weight, bias)

    out = pl.pallas_call(
        _apply_kernel,
        grid=(NSPLIT, steps_a),
        in_specs=[
            pl.BlockSpec((BB_A, HW, C), lambda i, j: (i * steps_a + j, 0, 0)),
            pl.BlockSpec((4, C), lambda i, j: (0, 0)),
            pl.BlockSpec((1, C), lambda i, j: (0, 0)),
        ],
        out_specs=pl.BlockSpec((BB_A, HW, C), lambda i, j: (i * steps_a + j, 0, 0)),
        out_shape=jax.ShapeDtypeStruct((B, HW, C), jnp.float32),
        compiler_params=pltpu.CompilerParams(
            dimension_semantics=("parallel", "arbitrary"),
            vmem_limit_bytes=56 * 1024 * 1024,
        ),
        name="qbn_apply",
    )(xt, amat, beff)

    return jnp.transpose(out.reshape(B, H, W, C), (0, 3, 1, 2))


# speedup vs baseline: 1.0126x; 1.0126x over previous
import functools

import jax
import jax.numpy as jnp
from jax.experimental import pallas as pl
from jax.experimental.pallas import tpu as pltpu

_EPS = 1e-5


def _rot(x, k):
    return jnp.concatenate([x[:, k:], x[:, :k]], axis=1)


def _stats_factor_kernel(x_ref, w_ref, b_ref, a_ref, beff_ref, mom_ref,
                         *, n_samples, n_steps):
    j = pl.program_id(0)

    @pl.when(j == 0)
    def _():
        mom_ref[...] = jnp.zeros_like(mom_ref)

    nb = x_ref.shape[0]
    acc = None
    for b in range(nb):
        xb = x_ref[b]
        r0 = jnp.sum(xb, axis=0, keepdims=True)
        r1 = jnp.sum(xb * xb, axis=0, keepdims=True)
        r2 = jnp.sum(xb * _rot(xb, 64), axis=0, keepdims=True)
        r3 = jnp.sum(xb * _rot(xb, 128), axis=0, keepdims=True)
        blk = jnp.concatenate([r0, r1, r2, r3], axis=0)
        acc = blk if acc is None else acc + blk
    mom_ref[...] += acc

    @pl.when(j == n_steps - 1)
    def _():
        _factor(mom_ref, w_ref, b_ref, a_ref, beff_ref, n_samples=n_samples)


def _factor(mom_ref, w_ref, b_ref, a_ref, beff_ref, *, n_samples):
    m = mom_ref[...]
    inv_n = 1.0 / float(n_samples)

    def seg(row, q):
        return m[row:row + 1, q * 64:(q + 1) * 64] * inv_n

    mean = [seg(0, q) for q in range(4)]
    a = {}
    for q in range(4):
        a[(q, q)] = seg(1, q) - mean[q] * mean[q] + _EPS
    a[(0, 1)] = seg(2, 0) - mean[0] * mean[1]
    a[(1, 2)] = seg(2, 1) - mean[1] * mean[2]
    a[(2, 3)] = seg(2, 2) - mean[2] * mean[3]
    a[(0, 3)] = seg(2, 3) - mean[0] * mean[3]
    a[(0, 2)] = seg(3, 0) - mean[0] * mean[2]
    a[(1, 3)] = seg(3, 1) - mean[1] * mean[3]

    def A(i, j):
        return a[(min(i, j), max(i, j))]

    l = {}
    l[(0, 0)] = jnp.sqrt(A(0, 0))
    inv00 = 1.0 / l[(0, 0)]
    l[(1, 0)] = A(1, 0) * inv00
    l[(2, 0)] = A(2, 0) * inv00
    l[(3, 0)] = A(3, 0) * inv00
    l[(1, 1)] = jnp.sqrt(A(1, 1) - l[(1, 0)] ** 2)
    inv11 = 1.0 / l[(1, 1)]
    l[(2, 1)] = (A(2, 1) - l[(2, 0)] * l[(1, 0)]) * inv11
    l[(3, 1)] = (A(3, 1) - l[(3, 0)] * l[(1, 0)]) * inv11
    l[(2, 2)] = jnp.sqrt(A(2, 2) - l[(2, 0)] ** 2 - l[(2, 1)] ** 2)
    inv22 = 1.0 / l[(2, 2)]
    l[(3, 2)] = (A(3, 2) - l[(3, 0)] * l[(2, 0)] - l[(3, 1)] * l[(2, 1)]) * inv22
    l[(3, 3)] = jnp.sqrt(A(3, 3) - l[(3, 0)] ** 2 - l[(3, 1)] ** 2 - l[(3, 2)] ** 2)
    inv33 = 1.0 / l[(3, 3)]

    kk = {(0, 0): inv00, (1, 1): inv11, (2, 2): inv22, (3, 3): inv33}
    kk[(1, 0)] = -(l[(1, 0)] * kk[(0, 0)]) * inv11
    kk[(2, 0)] = -(l[(2, 0)] * kk[(0, 0)] + l[(2, 1)] * kk[(1, 0)]) * inv22
    kk[(2, 1)] = -(l[(2, 1)] * kk[(1, 1)]) * inv22
    kk[(3, 0)] = -(l[(3, 0)] * kk[(0, 0)] + l[(3, 1)] * kk[(1, 0)]
                   + l[(3, 2)] * kk[(2, 0)]) * inv33
    kk[(3, 1)] = -(l[(3, 1)] * kk[(1, 1)] + l[(3, 2)] * kk[(2, 1)]) * inv33
    kk[(3, 2)] = -(l[(3, 2)] * kk[(2, 2)]) * inv33

    mm = {}
    beffs = []
    for p in range(4):
        wp = w_ref[p]
        w = [wp[r:r + 1, :] for r in range(4)]
        bp = b_ref[p:p + 1, :]
        for s in range(4):
            acc = w[0] * kk[(s, 0)]
            for r in range(1, s + 1):
                acc = acc + w[r] * kk[(s, r)]
            mm[(p, s)] = acc
            bp = bp - acc * mean[s]
        beffs.append(bp)

    rows = [jnp.concatenate([mm[(p, (p + s) % 4)] for p in range(4)], axis=1)
            for s in range(4)]
    a_ref[...] = jnp.concatenate(rows, axis=0)
    beff_ref[...] = jnp.concatenate(beffs, axis=1)


def _apply_kernel(x_ref, a_ref, beff_ref, o_ref):
    nb = x_ref.shape[0]
    a0 = a_ref[0:1, :]
    a1 = a_ref[1:2, :]
    a2 = a_ref[2:3, :]
    a3 = a_ref[3:4, :]
    bb = beff_ref[0:1, :]
    for b in range(nb):
        xb = x_ref[b]
        acc = bb + a0 * xb
        acc = acc + a1 * _rot(xb, 64)
        acc = acc + a2 * _rot(xb, 128)
        acc = acc + a3 * _rot(xb, 192)
        o_ref[b] = acc


def kernel(x, weight, bias):
    B, C, H, W = x.shape
    HW = H * W
    N = B * HW
    xt = jnp.transpose(x, (0, 2, 3, 1)).reshape(B, HW, C)

    NSPLIT = 2
    BB_S = 4
    BB_A = 4
    steps_s = B // BB_S
    steps_a = B // (BB_A * NSPLIT)

    amat, beff = pl.pallas_call(
        functools.partial(_stats_factor_kernel, n_samples=N, n_steps=steps_s),
        grid=(steps_s,),
        in_specs=[
            pl.BlockSpec((BB_S, HW, C), lambda j: (j, 0, 0)),
            pl.BlockSpec((4, 4, C // 4), lambda j: (0, 0, 0)),
            pl.BlockSpec((4, C // 4), lambda j: (0, 0)),
        ],
        out_specs=[
            pl.BlockSpec((4, C), lambda j: (0, 0)),
            pl.BlockSpec((1, C), lambda j: (0, 0)),
        ],
        out_shape=[
            jax.ShapeDtypeStruct((4, C), jnp.float32),
            jax.ShapeDtypeStruct((1, C), jnp.float32),
        ],
        scratch_shapes=[pltpu.VMEM((4, C), jnp.float32)],
        compiler_params=pltpu.CompilerParams(
            dimension_semantics=("arbitrary",),
            vmem_limit_bytes=56 * 1024 * 1024,
        ),
        name="qbn_stats_factor",
    )(xt, weight, bias)

    out = pl.pallas_call(
        _apply_kernel,
        grid=(NSPLIT, steps_a),
        in_specs=[
            pl.BlockSpec((BB_A, HW, C), lambda i, j: (i * steps_a + j, 0, 0)),
            pl.BlockSpec((4, C), lambda i, j: (0, 0)),
            pl.BlockSpec((1, C), lambda i, j: (0, 0)),
        ],
        out_specs=pl.BlockSpec((BB_A, HW, C), lambda i, j: (i * steps_a + j, 0, 0)),
        out_shape=jax.ShapeDtypeStruct((B, HW, C), jnp.float32),
        compiler_params=pltpu.CompilerParams(
            dimension_semantics=("parallel", "arbitrary"),
            vmem_limit_bytes=56 * 1024 * 1024,
        ),
        name="qbn_apply",
    )(xt, amat, beff)

    return jnp.transpose(out.reshape(B, H, W, C), (0, 3, 1, 2))
